# baseline (speedup 1.0000x reference)
import functools

import jax
import jax.numpy as jnp
from jax import lax
from jax.experimental import pallas as pl
from jax.experimental.pallas import tpu as pltpu
from jax.experimental.pallas import tpu_sc as plsc

N = 10000
E = 320000
D = 128
OUT = 10

NC = 2
NS = 16
NW = NC * NS
CSZ = 128
CH0 = 160
CH1 = 0
TOTCH = NS * (CH0 + CH1)
EPAD = TOTCH * CSZ
NPAD = 10112
RPW = NPAD // NS


G = 16


def _msg_body(h_hbm, src_hbm, dst_hbm, z_hbm, out_hbm,
              src_v, dst_v, rows_a, rows_b, table, sem_a, sem_b):
  cid = lax.axis_index("c")
  sid = lax.axis_index("s")
  r0 = sid * RPW
  base = jnp.where(cid == 0, sid * CH0, NS * CH0 + sid * CH1)
  nch = jnp.where(cid == 0, CH0, CH1)
  pltpu.sync_copy(z_hbm.at[pl.ds(r0, RPW)], table.at[pl.ds(r0, RPW)])
  plsc.subcore_barrier()

  @pl.loop(0, nch, step=G)
  def _(g):
    pltpu.sync_copy(src_hbm.at[pl.ds(base + g, G)], src_v)
    pltpu.sync_copy(dst_hbm.at[pl.ds(base + g, G)], dst_v)

    pltpu.async_copy(h_hbm.at[src_v.at[0]], rows_a, sem_a)

    @pl.loop(0, G, step=2)
    def _(i):
      pltpu.async_copy(h_hbm.at[src_v.at[i + 1]], rows_b, sem_b)
      pltpu.make_async_copy(h_hbm.at[src_v.at[i]], rows_a, sem_a).wait()
      pltpu.sync_copy(rows_a, table.at[dst_v.at[i]], add=True)

      @pl.when(i + 2 < G)
      def _():
        pltpu.async_copy(h_hbm.at[src_v.at[i + 2]], rows_a, sem_a)

      pltpu.make_async_copy(h_hbm.at[src_v.at[i + 1]], rows_b, sem_b).wait()
      pltpu.sync_copy(rows_b, table.at[dst_v.at[i + 1]], add=True)

  plsc.subcore_barrier()
  pltpu.sync_copy(table.at[pl.ds(r0, RPW)], out_hbm.at[cid, pl.ds(r0, RPW)])


@functools.lru_cache(maxsize=1)
def _get_sc_message_pass():
  mesh = plsc.VectorSubcoreMesh(core_axis_name="c", subcore_axis_name="s",
                                num_cores=NC, num_subcores=NS)
  return functools.partial(
      pl.kernel,
      out_type=jax.ShapeDtypeStruct((NC, NPAD, D), jnp.float32),
      mesh=mesh,
      scratch_types=[
          pltpu.VMEM((G, CSZ), jnp.int32),
          pltpu.VMEM((G, CSZ), jnp.int32),
          pltpu.VMEM((CSZ, D), jnp.float32),
          pltpu.VMEM((CSZ, D), jnp.float32),
          pltpu.VMEM_SHARED((NPAD, D), jnp.float32),
          pltpu.SemaphoreType.DMA,
          pltpu.SemaphoreType.DMA,
      ],
  )(_msg_body)


R = 1000


def _mlp_block(h_ref, p0_ref, p1_ref, w1_ref, b1_ref, w2_ref, b2_ref):
  agg = h_ref[...] + p0_ref[0] + p1_ref[0]
  z = jnp.dot(agg, w1_ref[...], preferred_element_type=jnp.float32)
  z = jnp.maximum(z + b1_ref[...], 0.0)
  o = jnp.dot(z, w2_ref[...], preferred_element_type=jnp.float32)
  return jnp.maximum(o + b2_ref[...], 0.0)


def _mlp_body(h_ref, p0_ref, p1_ref, w1_ref, b1_ref, w2_ref, b2_ref, o_ref):
  o_ref[...] = _mlp_block(h_ref, p0_ref, p1_ref, w1_ref, b1_ref, w2_ref,
                          b2_ref)


def _mlp_last_body(h_ref, p0_ref, p1_ref, w1_ref, b1_ref, w2_ref, b2_ref,
                   wo_ref, bo_ref, o_ref, gmax_ref):
  i = pl.program_id(0)
  o = _mlp_block(h_ref, p0_ref, p1_ref, w1_ref, b1_ref, w2_ref, b2_ref)
  bm = jnp.max(o, axis=0, keepdims=True)

  @pl.when(i == 0)
  def _():
    gmax_ref[...] = bm

  @pl.when(i > 0)
  def _():
    gmax_ref[...] = jnp.maximum(gmax_ref[...], bm)

  @pl.when(i == pl.num_programs(0) - 1)
  def _():
    g = gmax_ref[...]
    o_ref[...] = (jnp.dot(g, wo_ref[...], preferred_element_type=jnp.float32)
                  + bo_ref[...])


_p_spec0 = pl.BlockSpec((1, R, D), lambda i: (0, i, 0))
_p_spec1 = pl.BlockSpec((1, R, D), lambda i: (1, i, 0))
_h_spec = pl.BlockSpec((R, D), lambda i: (i, 0))
_w1_spec = pl.BlockSpec((D, 2 * D), lambda i: (0, 0))
_b1_spec = pl.BlockSpec((1, 2 * D), lambda i: (0, 0))
_w2_spec = pl.BlockSpec((2 * D, D), lambda i: (0, 0))
_b2_spec = pl.BlockSpec((1, D), lambda i: (0, 0))

_tc_mlp = pl.pallas_call(
    _mlp_body,
    grid=(N // R,),
    in_specs=[_h_spec, _p_spec0, _p_spec1, _w1_spec, _b1_spec, _w2_spec,
              _b2_spec],
    out_specs=_h_spec,
    out_shape=jax.ShapeDtypeStruct((N, D), jnp.float32),
)

_tc_mlp_last = pl.pallas_call(
    _mlp_last_body,
    grid=(N // R,),
    in_specs=[_h_spec, _p_spec0, _p_spec1, _w1_spec, _b1_spec, _w2_spec,
              _b2_spec,
              pl.BlockSpec((D, OUT), lambda i: (0, 0)),
              pl.BlockSpec((1, OUT), lambda i: (0, 0))],
    out_specs=pl.BlockSpec((1, OUT), lambda i: (0, 0)),
    out_shape=jax.ShapeDtypeStruct((1, OUT), jnp.float32),
    scratch_shapes=[pltpu.VMEM((1, D), jnp.float32)],
)


def kernel(x, edge_index, W1_0, b1_0, W2_0, b2_0, W1_1, b1_1, W2_1, b2_1,
           W1_2, b1_2, W2_2, b2_2, W_out, b_out):
  src = edge_index[0]
  dst = edge_index[1]
  pad = EPAD - E
  srcp = jnp.concatenate([src, jnp.zeros((pad,), jnp.int32)])
  dstp = jnp.concatenate([dst, jnp.full((pad,), N, jnp.int32)])
  srcp = srcp.reshape(TOTCH, CSZ)
  dstp = dstp.reshape(TOTCH, CSZ)
  zeros = jnp.zeros((NPAD, D), jnp.float32)

  params = [(W1_0, b1_0, W2_0, b2_0), (W1_1, b1_1, W2_1, b2_1),
            (W1_2, b1_2, W2_2, b2_2)]
  sc_message_pass = _get_sc_message_pass()
  h = x
  for l, (W1, b1, W2, b2) in enumerate(params):
    p = sc_message_pass(h, srcp, dstp, zeros)
    if l < 2:
      h = _tc_mlp(h, p, p, W1, b1.reshape(1, -1), W2, b2.reshape(1, -1))
    else:
      out = _tc_mlp_last(h, p, p, W1, b1.reshape(1, -1), W2,
                         b2.reshape(1, -1), W_out, b_out.reshape(1, -1))
  return out


# speedup vs baseline: 3.7702x; 3.7702x over previous
import functools

import jax
import jax.numpy as jnp
from jax import lax
from jax.experimental import pallas as pl
from jax.experimental.pallas import tpu as pltpu
from jax.experimental.pallas import tpu_sc as plsc

N = 10000
E = 320000
D = 128
OUT = 10

NC = 2
NS = 16
NW = NC * NS
CSZ = 128
CH0 = 80
CH1 = 80
TOTCH = NS * (CH0 + CH1)
EPAD = TOTCH * CSZ
NPAD = 10112
RPW = NPAD // NS


G = 16


def _msg_body(h_hbm, src_hbm, dst_hbm, z_hbm, out_hbm,
              src_v, dst_v, rows_a, rows_b, table, sem_a, sem_b):
  cid = lax.axis_index("c")
  sid = lax.axis_index("s")
  r0 = sid * RPW
  base = jnp.where(cid == 0, sid * CH0, NS * CH0 + sid * CH1)
  nch = jnp.where(cid == 0, CH0, CH1)
  pltpu.sync_copy(z_hbm.at[pl.ds(r0, RPW)], table.at[pl.ds(r0, RPW)])
  plsc.subcore_barrier()

  @pl.loop(0, nch, step=G)
  def _(g):
    pltpu.sync_copy(src_hbm.at[pl.ds(base + g, G)], src_v)
    pltpu.sync_copy(dst_hbm.at[pl.ds(base + g, G)], dst_v)

    pltpu.async_copy(h_hbm.at[src_v.at[0]], rows_a, sem_a)

    @pl.loop(0, G, step=2)
    def _(i):
      pltpu.async_copy(h_hbm.at[src_v.at[i + 1]], rows_b, sem_b)
      pltpu.make_async_copy(h_hbm.at[src_v.at[i]], rows_a, sem_a).wait()
      pltpu.sync_copy(rows_a, table.at[dst_v.at[i]], add=True)

      @pl.when(i + 2 < G)
      def _():
        pltpu.async_copy(h_hbm.at[src_v.at[i + 2]], rows_a, sem_a)

      pltpu.make_async_copy(h_hbm.at[src_v.at[i + 1]], rows_b, sem_b).wait()
      pltpu.sync_copy(rows_b, table.at[dst_v.at[i + 1]], add=True)

  plsc.subcore_barrier()
  pltpu.sync_copy(table.at[pl.ds(r0, RPW)], out_hbm.at[cid, pl.ds(r0, RPW)])


@functools.lru_cache(maxsize=1)
def _get_sc_message_pass():
  mesh = plsc.VectorSubcoreMesh(core_axis_name="c", subcore_axis_name="s",
                                num_cores=NC, num_subcores=NS)
  return functools.partial(
      pl.kernel,
      out_type=jax.ShapeDtypeStruct((NC, NPAD, D), jnp.float32),
      mesh=mesh,
      scratch_types=[
          pltpu.VMEM((G, CSZ), jnp.int32),
          pltpu.VMEM((G, CSZ), jnp.int32),
          pltpu.VMEM((CSZ, D), jnp.float32),
          pltpu.VMEM((CSZ, D), jnp.float32),
          pltpu.VMEM_SHARED((NPAD, D), jnp.float32),
          pltpu.SemaphoreType.DMA,
          pltpu.SemaphoreType.DMA,
      ],
  )(_msg_body)


R = 1000


def _mlp_block(h_ref, p0_ref, p1_ref, w1_ref, b1_ref, w2_ref, b2_ref):
  agg = h_ref[...] + p0_ref[0] + p1_ref[0]
  z = jnp.dot(agg, w1_ref[...], preferred_element_type=jnp.float32)
  z = jnp.maximum(z + b1_ref[...], 0.0)
  o = jnp.dot(z, w2_ref[...], preferred_element_type=jnp.float32)
  return jnp.maximum(o + b2_ref[...], 0.0)


def _mlp_body(h_ref, p0_ref, p1_ref, w1_ref, b1_ref, w2_ref, b2_ref, o_ref):
  o_ref[...] = _mlp_block(h_ref, p0_ref, p1_ref, w1_ref, b1_ref, w2_ref,
                          b2_ref)


def _mlp_last_body(h_ref, p0_ref, p1_ref, w1_ref, b1_ref, w2_ref, b2_ref,
                   wo_ref, bo_ref, o_ref, gmax_ref):
  i = pl.program_id(0)
  o = _mlp_block(h_ref, p0_ref, p1_ref, w1_ref, b1_ref, w2_ref, b2_ref)
  bm = jnp.max(o, axis=0, keepdims=True)

  @pl.when(i == 0)
  def _():
    gmax_ref[...] = bm

  @pl.when(i > 0)
  def _():
    gmax_ref[...] = jnp.maximum(gmax_ref[...], bm)

  @pl.when(i == pl.num_programs(0) - 1)
  def _():
    g = gmax_ref[...]
    o_ref[...] = (jnp.dot(g, wo_ref[...], preferred_element_type=jnp.float32)
                  + bo_ref[...])


_p_spec0 = pl.BlockSpec((1, R, D), lambda i: (0, i, 0))
_p_spec1 = pl.BlockSpec((1, R, D), lambda i: (1, i, 0))
_h_spec = pl.BlockSpec((R, D), lambda i: (i, 0))
_w1_spec = pl.BlockSpec((D, 2 * D), lambda i: (0, 0))
_b1_spec = pl.BlockSpec((1, 2 * D), lambda i: (0, 0))
_w2_spec = pl.BlockSpec((2 * D, D), lambda i: (0, 0))
_b2_spec = pl.BlockSpec((1, D), lambda i: (0, 0))

_tc_mlp = pl.pallas_call(
    _mlp_body,
    grid=(N // R,),
    in_specs=[_h_spec, _p_spec0, _p_spec1, _w1_spec, _b1_spec, _w2_spec,
              _b2_spec],
    out_specs=_h_spec,
    out_shape=jax.ShapeDtypeStruct((N, D), jnp.float32),
)

_tc_mlp_last = pl.pallas_call(
    _mlp_last_body,
    grid=(N // R,),
    in_specs=[_h_spec, _p_spec0, _p_spec1, _w1_spec, _b1_spec, _w2_spec,
              _b2_spec,
              pl.BlockSpec((D, OUT), lambda i: (0, 0)),
              pl.BlockSpec((1, OUT), lambda i: (0, 0))],
    out_specs=pl.BlockSpec((1, OUT), lambda i: (0, 0)),
    out_shape=jax.ShapeDtypeStruct((1, OUT), jnp.float32),
    scratch_shapes=[pltpu.VMEM((1, D), jnp.float32)],
)


def kernel(x, edge_index, W1_0, b1_0, W2_0, b2_0, W1_1, b1_1, W2_1, b2_1,
           W1_2, b1_2, W2_2, b2_2, W_out, b_out):
  src = edge_index[0]
  dst = edge_index[1]
  pad = EPAD - E
  pad_iota = jnp.arange(pad, dtype=jnp.int32)
  srcp = jnp.concatenate([src, pad_iota % N])
  dstp = jnp.concatenate([dst, N + pad_iota % (NPAD - N)])
  srcp = srcp.reshape(TOTCH, CSZ)
  dstp = dstp.reshape(TOTCH, CSZ)
  zeros = jnp.zeros((NPAD, D), jnp.float32)

  params = [(W1_0, b1_0, W2_0, b2_0), (W1_1, b1_1, W2_1, b2_1),
            (W1_2, b1_2, W2_2, b2_2)]
  sc_message_pass = _get_sc_message_pass()
  h = x
  for l, (W1, b1, W2, b2) in enumerate(params):
    p = sc_message_pass(h, srcp, dstp, zeros)
    if l < 2:
      h = _tc_mlp(h, p, p, W1, b1.reshape(1, -1), W2, b2.reshape(1, -1))
    else:
      out = _tc_mlp_last(h, p, p, W1, b1.reshape(1, -1), W2,
                         b2.reshape(1, -1), W_out, b_out.reshape(1, -1))
  return out


# speedup vs baseline: 4.0584x; 1.0764x over previous
import functools

import jax
import jax.numpy as jnp
from jax import lax
from jax.experimental import pallas as pl
from jax.experimental.pallas import tpu as pltpu
from jax.experimental.pallas import tpu_sc as plsc

N = 10000
E = 320000
D = 128
OUT = 10

NC = 2
NS = 16
NW = NC * NS
CSZ = 112
CH0 = 90
CH1 = 90
TOTCH = NS * (CH0 + CH1)
EPAD = TOTCH * CSZ
NPAD = 10112
RPW = NPAD // NS

G = 15
GPW = CH0 // G
NG = TOTCH // G


def _msg_body(h_hbm, src_hbm, dst_hbm, z_hbm, out_hbm,
              src_v, dst_v, rows_a, rows_b, rows_c, table,
              sem_a, sem_b, sem_c):
  cid = lax.axis_index("c")
  sid = lax.axis_index("s")
  r0 = sid * RPW
  base = (cid * NS + sid) * GPW
  pltpu.sync_copy(z_hbm.at[pl.ds(r0, RPW)], table.at[pl.ds(r0, RPW)])
  plsc.subcore_barrier()

  @pl.loop(0, GPW)
  def _(g):
    pltpu.sync_copy(src_hbm.at[base + g], src_v)
    pltpu.sync_copy(dst_hbm.at[base + g], dst_v)

    pltpu.async_copy(h_hbm.at[src_v.at[0]], rows_a, sem_a)
    pltpu.async_copy(h_hbm.at[src_v.at[1]], rows_b, sem_b)

    @pl.loop(0, G, step=3)
    def _(i):
      pltpu.make_async_copy(h_hbm.at[src_v.at[i]], rows_a, sem_a).wait()

      @pl.when(i + 2 < G)
      def _():
        pltpu.async_copy(h_hbm.at[src_v.at[i + 2]], rows_c, sem_c)

      pltpu.sync_copy(rows_a, table.at[dst_v.at[i]], add=True)

      pltpu.make_async_copy(h_hbm.at[src_v.at[i + 1]], rows_b, sem_b).wait()

      @pl.when(i + 3 < G)
      def _():
        pltpu.async_copy(h_hbm.at[src_v.at[i + 3]], rows_a, sem_a)

      pltpu.sync_copy(rows_b, table.at[dst_v.at[i + 1]], add=True)

      pltpu.make_async_copy(h_hbm.at[src_v.at[i + 2]], rows_c, sem_c).wait()

      @pl.when(i + 4 < G)
      def _():
        pltpu.async_copy(h_hbm.at[src_v.at[i + 4]], rows_b, sem_b)

      pltpu.sync_copy(rows_c, table.at[dst_v.at[i + 2]], add=True)

  plsc.subcore_barrier()
  pltpu.sync_copy(table.at[pl.ds(r0, RPW)], out_hbm.at[cid, pl.ds(r0, RPW)])


@functools.lru_cache(maxsize=1)
def _get_sc_message_pass():
  mesh = plsc.VectorSubcoreMesh(core_axis_name="c", subcore_axis_name="s",
                                num_cores=NC, num_subcores=NS)
  return functools.partial(
      pl.kernel,
      out_type=jax.ShapeDtypeStruct((NC, NPAD, D), jnp.float32),
      mesh=mesh,
      scratch_types=[
          pltpu.VMEM((G, CSZ), jnp.int32),
          pltpu.VMEM((G, CSZ), jnp.int32),
          pltpu.VMEM((CSZ, D), jnp.float32),
          pltpu.VMEM((CSZ, D), jnp.float32),
          pltpu.VMEM((CSZ, D), jnp.float32),
          pltpu.VMEM_SHARED((NPAD, D), jnp.float32),
          pltpu.SemaphoreType.DMA,
          pltpu.SemaphoreType.DMA,
          pltpu.SemaphoreType.DMA,
      ],
  )(_msg_body)


R = 1000


def _mlp_block(h_ref, p0_ref, p1_ref, w1_ref, b1_ref, w2_ref, b2_ref):
  agg = h_ref[...] + p0_ref[0] + p1_ref[0]
  z = jnp.dot(agg, w1_ref[...], preferred_element_type=jnp.float32)
  z = jnp.maximum(z + b1_ref[...], 0.0)
  o = jnp.dot(z, w2_ref[...], preferred_element_type=jnp.float32)
  return jnp.maximum(o + b2_ref[...], 0.0)


def _mlp_body(h_ref, p0_ref, p1_ref, w1_ref, b1_ref, w2_ref, b2_ref, o_ref):
  o_ref[...] = _mlp_block(h_ref, p0_ref, p1_ref, w1_ref, b1_ref, w2_ref,
                          b2_ref)


def _mlp_last_body(h_ref, p0_ref, p1_ref, w1_ref, b1_ref, w2_ref, b2_ref,
                   wo_ref, bo_ref, o_ref, gmax_ref):
  i = pl.program_id(0)
  o = _mlp_block(h_ref, p0_ref, p1_ref, w1_ref, b1_ref, w2_ref, b2_ref)
  bm = jnp.max(o, axis=0, keepdims=True)

  @pl.when(i == 0)
  def _():
    gmax_ref[...] = bm

  @pl.when(i > 0)
  def _():
    gmax_ref[...] = jnp.maximum(gmax_ref[...], bm)

  @pl.when(i == pl.num_programs(0) - 1)
  def _():
    g = gmax_ref[...]
    o_ref[...] = (jnp.dot(g, wo_ref[...], preferred_element_type=jnp.float32)
                  + bo_ref[...])


_p_spec0 = pl.BlockSpec((1, R, D), lambda i: (0, i, 0))
_p_spec1 = pl.BlockSpec((1, R, D), lambda i: (1, i, 0))
_h_spec = pl.BlockSpec((R, D), lambda i: (i, 0))
_w1_spec = pl.BlockSpec((D, 2 * D), lambda i: (0, 0))
_b1_spec = pl.BlockSpec((1, 2 * D), lambda i: (0, 0))
_w2_spec = pl.BlockSpec((2 * D, D), lambda i: (0, 0))
_b2_spec = pl.BlockSpec((1, D), lambda i: (0, 0))

_tc_mlp = pl.pallas_call(
    _mlp_body,
    grid=(N // R,),
    in_specs=[_h_spec, _p_spec0, _p_spec1, _w1_spec, _b1_spec, _w2_spec,
              _b2_spec],
    out_specs=_h_spec,
    out_shape=jax.ShapeDtypeStruct((N, D), jnp.float32),
)

_tc_mlp_last = pl.pallas_call(
    _mlp_last_body,
    grid=(N // R,),
    in_specs=[_h_spec, _p_spec0, _p_spec1, _w1_spec, _b1_spec, _w2_spec,
              _b2_spec,
              pl.BlockSpec((D, OUT), lambda i: (0, 0)),
              pl.BlockSpec((1, OUT), lambda i: (0, 0))],
    out_specs=pl.BlockSpec((1, OUT), lambda i: (0, 0)),
    out_shape=jax.ShapeDtypeStruct((1, OUT), jnp.float32),
    scratch_shapes=[pltpu.VMEM((1, D), jnp.float32)],
)


def kernel(x, edge_index, W1_0, b1_0, W2_0, b2_0, W1_1, b1_1, W2_1, b2_1,
           W1_2, b1_2, W2_2, b2_2, W_out, b_out):
  src = edge_index[0]
  dst = edge_index[1]
  pad = EPAD - E
  pad_iota = jnp.arange(pad, dtype=jnp.int32)
  srcp = jnp.concatenate([src, pad_iota % N])
  dstp = jnp.concatenate([dst, N + pad_iota % (NPAD - N)])
  srcp = srcp.reshape(NG, G, CSZ)
  dstp = dstp.reshape(NG, G, CSZ)
  zeros = jnp.zeros((NPAD, D), jnp.float32)

  params = [(W1_0, b1_0, W2_0, b2_0), (W1_1, b1_1, W2_1, b2_1),
            (W1_2, b1_2, W2_2, b2_2)]
  sc_message_pass = _get_sc_message_pass()
  h = x
  for l, (W1, b1, W2, b2) in enumerate(params):
    p = sc_message_pass(h, srcp, dstp, zeros)
    if l < 2:
      h = _tc_mlp(h, p, p, W1, b1.reshape(1, -1), W2, b2.reshape(1, -1))
    else:
      out = _tc_mlp_last(h, p, p, W1, b1.reshape(1, -1), W2,
                         b2.reshape(1, -1), W_out, b_out.reshape(1, -1))
  return out
